# Initial kernel scaffold; baseline (speedup 1.0000x reference)
import jax
import jax.numpy as jnp
from jax.experimental import pallas as pl


def kernel(Z, edge_index):
    raise NotImplementedError("write your pallas kernel here")


# speedup vs baseline: 3.8116x; 3.8116x over previous
import functools

import jax
import jax.numpy as jnp
from jax import lax
from jax.experimental import pallas as pl
from jax.experimental.pallas import tpu as pltpu
from jax.experimental.pallas import tpu_sc as plsc

NC = 2
NS = 16
L = 16
B = 80
BF = 40


def _sc_body(N, CH, n_echunk, n_fchunk,
             zs, colb_h, rowb_h, outl, outh,
             colv, rowv, gbuf, onesb, degb,
             acc, deg, sem):
    cid = lax.axis_index("c")
    sid = lax.axis_index("s")
    w = cid * NS + sid
    zero16 = jnp.zeros((L,), jnp.float32)
    ones16 = jnp.ones((L,), jnp.float32)

    def init_row(r, carry):
        for c8 in range(CH // L):
            gbuf[r, pl.ds(c8 * L, L)] = zero16
        return carry
    lax.fori_loop(0, 2 * BF, init_row, 0)

    def init_small(r, carry):
        onesb[pl.ds(r * L, L)] = zero16
        return carry
    lax.fori_loop(0, B // L, init_small, 0)

    def zero_chunk(c, carry):
        ch = sid + NS * c
        @pl.when(ch < N // B)
        def _():
            pltpu.sync_copy(gbuf, acc.at[pl.ds(ch * B, B)])
            pltpu.sync_copy(onesb, deg.at[pl.ds(ch * B, B)])
        return carry
    lax.fori_loop(0, (N // B + NS - 1) // NS, zero_chunk, 0)

    def ones_row(r, carry):
        onesb[pl.ds(r * L, L)] = ones16
        return carry
    lax.fori_loop(0, B // L, ones_row, 0)
    plsc.subcore_barrier()

    def edge_chunk(k, carry):
        pltpu.sync_copy(colb_h.at[w, k], colv)
        pltpu.sync_copy(rowb_h.at[sid, k], rowv)
        pltpu.async_copy(zs.at[colv.at[0]], gbuf, sem).wait()
        pltpu.sync_copy(gbuf, acc.at[rowv.at[0]], add=True)
        pltpu.sync_copy(onesb, deg.at[rowv.at[0]], add=True)
        return carry
    lax.fori_loop(0, n_echunk, edge_chunk, 0)
    plsc.subcore_barrier()

    g_acc = gbuf.at[pl.ds(0, BF)]
    g_z = gbuf.at[pl.ds(BF, BF)]
    groups = []
    r0 = 0
    while r0 < BF:
        groups.append((r0, min(L, BF - r0)))
        r0 += L

    def fin_chunk(c, carry):
        ch = sid + NS * c
        @pl.when(ch < n_fchunk)
        def _():
            base = ch * BF
            pltpu.sync_copy(acc.at[pl.ds(base, BF)], g_acc)
            pltpu.sync_copy(zs.at[pl.ds(cid * N + base, BF)], g_z)
            pltpu.sync_copy(deg.at[pl.ds(base, BF)], degb.at[pl.ds(0, BF)])

            for g0, nrows in groups:
                dv = degb[pl.ds(g0, L)]
                rdv = 1.0 / jnp.where(dv == 0.0, 1.0, dv)
                for l in range(nrows):
                    r = g0 + l
                    rd = rdv[l]
                    for c8 in range(CH // L):
                        sl = pl.ds(c8 * L, L)
                        zl = gbuf[r, sl] * rd
                        gbuf[r, sl] = zl
                        gbuf[BF + r, sl] = gbuf[BF + r, sl] - zl

            pltpu.sync_copy(g_acc, outl.at[pl.ds(cid * N + base, BF)])
            pltpu.sync_copy(g_z, outh.at[pl.ds(cid * N + base, BF)])
        return carry
    lax.fori_loop(0, (n_fchunk + NS - 1) // NS, fin_chunk, 0)


def kernel(Z, edge_index):
    N, C = Z.shape
    E = edge_index.shape[1]
    CH = C // NC
    n_echunk = E // (NS * B)
    n_fchunk = N // BF

    row = edge_index[0]
    col = edge_index[1]
    zs = Z.reshape(N, NC, CH).transpose(1, 0, 2).reshape(NC * N, CH)
    col2 = jnp.concatenate([col, col + N]).reshape(NC * NS, n_echunk, 1, B)
    row2 = row.reshape(NS, n_echunk, 1, B)

    body = functools.partial(_sc_body, N, CH, n_echunk, n_fchunk)
    mesh = plsc.VectorSubcoreMesh(core_axis_name="c", subcore_axis_name="s")
    outl, outh = pl.kernel(
        body,
        out_type=(
            jax.ShapeDtypeStruct((NC * N, CH), jnp.float32),
            jax.ShapeDtypeStruct((NC * N, CH), jnp.float32),
        ),
        mesh=mesh,
        scratch_types=(
            pltpu.VMEM((1, B), jnp.int32),
            pltpu.VMEM((1, B), jnp.int32),
            pltpu.VMEM((2 * BF, CH), jnp.float32),
            pltpu.VMEM((B,), jnp.float32),
            pltpu.VMEM((((BF + L - 1) // L) * L,), jnp.float32),
            pltpu.VMEM_SHARED((N, CH), jnp.float32),
            pltpu.VMEM_SHARED((N,), jnp.float32),
            pltpu.SemaphoreType.DMA,
        ),
        name="spectral_decomposer_sc",
    )(zs, col2, row2)

    z_low = outl.reshape(NC, N, CH).transpose(1, 0, 2).reshape(N, C)
    z_high = outh.reshape(NC, N, CH).transpose(1, 0, 2).reshape(N, C)
    return (z_low, z_high)
